# Initial kernel scaffold; baseline (speedup 1.0000x reference)
import jax
import jax.numpy as jnp
from jax.experimental import pallas as pl


def kernel(atomic_numbers, positions, Ef, dst_idx, src_idx, embed, element_bias, W_mp1, w_r1, W_mp2, w_r2, W_d1, W_d2, W_td, W_ro, w_out):
    raise NotImplementedError("write your pallas kernel here")


# speedup vs baseline: 32.9688x; 32.9688x over previous
import jax
import jax.numpy as jnp
from jax.experimental import pallas as pl
from math import comb

F = 32
K = 8
L = 9
Q = 2 * L
NITER = 3
CUTOFF = 5.0
ZMAX = 56
MB = 2

_BINOM = [float(comb(K - 1, k)) for k in range(K)]


def _gated_silu(h):
    s0 = h[..., 0:1, :]
    r0 = h[..., 1:L, :]
    s1 = h[..., L:L + 1, :]
    r1 = h[..., L + 1:, :]
    g0 = jax.nn.sigmoid(s0)
    g1 = jax.nn.sigmoid(s1)
    return jnp.concatenate([s0 * g0, r0 * g0, s1 * g1, r1 * g1], axis=-2)


def _l0_broadcast(h):
    shape = h.shape[:-2] + (L, F)
    b0 = jnp.broadcast_to(h[..., 0:1, :], shape)
    b1 = jnp.broadcast_to(h[..., L:L + 1, :], shape)
    return jnp.concatenate([b0, b1], axis=-2)


def _mp_kernel(an_ref, pos_ref, ef_ref, dst_ref, src_ref, embed_ref, bias_ref,
               Wmp1_ref, wr1_ref, Wmp2_ref, wr2_ref, Wd1_ref, Wd2_ref,
               Wtd_ref, Wro_ref, wout_ref, out_ref):
    E = dst_ref.shape[0]
    N = an_ref.shape[2]
    pos = pos_ref[...]
    an = an_ref[...][:, 0, :]
    ef = ef_ref[...][:, 0, :]
    dst = dst_ref[...]
    src = src_ref[...]

    iota_n = jax.lax.broadcasted_iota(jnp.int32, (E, N), 1)
    D = (dst == iota_n).astype(jnp.float32)
    S = (src == iota_n).astype(jnp.float32)

    psrc = jnp.einsum('en,bnc->bec', S, pos)
    pdst = jnp.einsum('en,bnc->bec', D, pos)
    disp = psrc - pdst
    r = jnp.sqrt(jnp.sum(disp * disp, axis=-1) + 1e-12)
    u = disp / r[..., None]
    ux = u[..., 0]
    uy = u[..., 1]
    uz = u[..., 2]
    Ylist = [None, ux, uy, uz, ux * uy, uy * uz, 3.0 * uz * uz - 1.0,
             ux * uz, ux * ux - uy * uy]

    t = r / (r + 1.0)
    om = 1.0 - t
    tp = [jnp.ones_like(t)]
    op = [jnp.ones_like(t)]
    for _ in range(1, K):
        tp.append(tp[-1] * t)
        op.append(op[-1] * om)
    x2 = (r / CUTOFF) ** 2
    c = jnp.where(x2 < 1.0,
                  jnp.exp(1.0 - 1.0 / jnp.maximum(1.0 - x2, 1e-9)), 0.0)
    basis = [_BINOM[k] * tp[k] * op[K - 1 - k] * c for k in range(K)]

    wr1 = wr1_ref[...]
    wr2 = wr2_ref[...]

    St = S

    def make_M(w):
        Dw = jnp.transpose(D)[None, :, :] * w[:, None, :]
        return jnp.einsum('bne,em->bnm', Dw, St)

    zoh = (an[..., None] == jax.lax.broadcasted_iota(
        jnp.int32, an.shape + (ZMAX,), 2)).astype(jnp.float32)
    e0 = jnp.einsum('bnz,zf->bnf', zoh, embed_ref[...])
    x = jnp.pad(e0[:, :, None, :], ((0, 0), (0, 0), (0, Q - 1), (0, 0)))

    ef4 = jnp.pad(ef, ((0, 0), (1, 0)))
    efL = jnp.pad(ef4, ((0, 0), (0, L - 4)))
    efQ = jnp.concatenate([efL, efL], axis=1)
    xEF = jnp.broadcast_to(efQ[:, None, :, None], x.shape)

    qi = jax.lax.broadcasted_iota(jnp.int32, (1, 1, Q, 1), 2)
    mask_l0 = jnp.where((qi == 0) | (qi == L), 1.0, 0.0)

    for i in range(NITER):
        rs1 = basis[0] * wr1[i, 0]
        rs2 = basis[0] * wr2[i, 0]
        for k in range(1, K):
            rs1 = rs1 + basis[k] * wr1[i, k]
            rs2 = rs2 + basis[k] * wr2[i, k]

        xw1 = jnp.einsum('bnqf,fg->bnqg', x, Wmp1_ref[i])
        y = jnp.einsum('bnm,bmqf->bnqf', make_M(rs1), xw1)

        x00 = x[:, :, 0, :]
        xw2 = jnp.einsum('bnf,fg->bng', x00, Wmp2_ref[i])
        if i < NITER - 1:
            contribs = [jnp.einsum('bnm,bmf->bnf', make_M(rs2), xw2)]
            for l in range(1, L):
                contribs.append(jnp.einsum('bnm,bmf->bnf',
                                           make_M(Ylist[l] * rs2), xw2))
            zeroL = jnp.zeros(x.shape[:2] + (L, F), dtype=x.dtype)
            yY = jnp.concatenate(
                [cc[:, :, None, :] for cc in contribs] + [zeroL], axis=2)
            y = y + yY
        else:
            y = y * mask_l0
            c0 = jnp.einsum('bnm,bmf->bnf', make_M(rs2), xw2)
            y = y + jnp.pad(c0[:, :, None, :],
                            ((0, 0), (0, 0), (0, Q - 1), (0, 0)))

        x = x + y
        h = _gated_silu(jnp.einsum('bnqf,fg->bnqg', x, Wd1_ref[i]))
        h = jnp.einsum('bnqf,fg->bnqg', h, Wd2_ref[i])
        x = h + y
        x = _gated_silu(x)
        tt = x * _l0_broadcast(xEF) + xEF * _l0_broadcast(x)
        x = x + tt
        x = jnp.einsum('bnqf,fg->bnqg', x, Wtd_ref[i])
        x = x + tt
        x = _gated_silu(x)
        xEF = tt

    x = _gated_silu(x)
    x0 = x[:, :, 0, :]
    for j in range(3):
        h = jnp.einsum('bnf,fg->bng', x0, Wro_ref[j])
        x0 = h * jax.nn.sigmoid(h)
    bias = jnp.einsum('bnz,zo->bno', zoh, bias_ref[...])[..., 0]
    ae = jnp.einsum('bnf,fo->bno', x0, wout_ref[...])[..., 0] + bias
    out_ref[...] = jnp.sum(ae, axis=1)[:, None, None]


def kernel(atomic_numbers, positions, Ef, dst_idx, src_idx, embed,
           element_bias, W_mp1, w_r1, W_mp2, w_r2, W_d1, W_d2, W_td, W_ro,
           w_out):
    B, N = atomic_numbers.shape
    E = dst_idx.shape[0]
    an3 = atomic_numbers.reshape(B, 1, N)
    ef3 = Ef.reshape(B, 1, 3)
    dst2 = dst_idx.reshape(E, 1).astype(jnp.int32)
    src2 = src_idx.reshape(E, 1).astype(jnp.int32)
    bias2 = element_bias.reshape(ZMAX, 1)
    wout2 = w_out.reshape(F, 1)

    grid = (B // MB,)
    full = lambda shp: pl.BlockSpec(shp, lambda i: (0,) * len(shp))
    out = pl.pallas_call(
        _mp_kernel,
        grid=grid,
        in_specs=[
            pl.BlockSpec((MB, 1, N), lambda i: (i, 0, 0)),
            pl.BlockSpec((MB, N, 3), lambda i: (i, 0, 0)),
            pl.BlockSpec((MB, 1, 3), lambda i: (i, 0, 0)),
            full((E, 1)),
            full((E, 1)),
            full((ZMAX, F)),
            full((ZMAX, 1)),
            full((NITER, F, F)),
            full((NITER, K)),
            full((NITER, F, F)),
            full((NITER, K)),
            full((NITER, F, F)),
            full((NITER, F, F)),
            full((NITER, F, F)),
            full((3, F, F)),
            full((F, 1)),
        ],
        out_specs=pl.BlockSpec((MB, 1, 1), lambda i: (i, 0, 0)),
        out_shape=jax.ShapeDtypeStruct((B, 1, 1), jnp.float32),
    )(an3, positions, ef3, dst2, src2, embed, bias2,
      W_mp1, w_r1, W_mp2, w_r2, W_d1, W_d2, W_td, W_ro, wout2)
    return out[:, 0, 0]


# speedup vs baseline: 32.9928x; 1.0007x over previous
import jax
import jax.numpy as jnp
import functools
from jax.experimental import pallas as pl
from math import comb

F = 32
K = 8
L = 9
Q = 2 * L
NITER = 3
CUTOFF = 5.0
ZMAX = 56
MB = 2

_BINOM = [float(comb(K - 1, k)) for k in range(K)]
_es = jnp.einsum


def _sigmoid(x):
    return 0.5 * (jnp.tanh(0.5 * x) + 1.0)


def _gated_silu(h):
    s0 = h[..., 0:1, :]
    r0 = h[..., 1:L, :]
    s1 = h[..., L:L + 1, :]
    r1 = h[..., L + 1:, :]
    g0 = _sigmoid(s0)
    g1 = _sigmoid(s1)
    return jnp.concatenate([s0 * g0, r0 * g0, s1 * g1, r1 * g1], axis=-2)


def _l0_broadcast(h):
    shape = h.shape[:-2] + (L, F)
    b0 = jnp.broadcast_to(h[..., 0:1, :], shape)
    b1 = jnp.broadcast_to(h[..., L:L + 1, :], shape)
    return jnp.concatenate([b0, b1], axis=-2)


def _mp_kernel(an_ref, pos_ref, ef_ref, dst_ref, src_ref, embed_ref, bias_ref,
               Wmp1_ref, wr1_ref, Wmp2_ref, wr2_ref, Wd1_ref, Wd2_ref,
               Wtd_ref, Wro_ref, wout_ref, out_ref):
    E = dst_ref.shape[0]
    N = an_ref.shape[2]
    pos = pos_ref[...]
    an = an_ref[...][:, 0, :]
    ef = ef_ref[...][:, 0, :]
    dst = dst_ref[...]
    src = src_ref[...]

    iota_n = jax.lax.broadcasted_iota(jnp.int32, (E, N), 1)
    D = (dst == iota_n).astype(jnp.float32)
    S = (src == iota_n).astype(jnp.float32)

    psrc = _es('en,bnc->bec', S, pos)
    pdst = _es('en,bnc->bec', D, pos)
    disp = psrc - pdst
    r = jnp.sqrt(jnp.sum(disp * disp, axis=-1) + 1e-12)
    u = disp / r[..., None]
    ux = u[..., 0]
    uy = u[..., 1]
    uz = u[..., 2]
    Ylist = [None, ux, uy, uz, ux * uy, uy * uz, 3.0 * uz * uz - 1.0,
             ux * uz, ux * ux - uy * uy]

    t = r / (r + 1.0)
    om = 1.0 - t
    tp = [jnp.power(t, float(k)) for k in range(K)]
    op = [jnp.power(om, float(k)) for k in range(K)]
    x2 = (r / CUTOFF) ** 2
    c = jnp.where(x2 < 1.0,
                  jnp.exp(1.0 - 1.0 / jnp.maximum(1.0 - x2, 1e-9)), 0.0)
    basis = [_BINOM[k] * tp[k] * op[K - 1 - k] * c for k in range(K)]

    wr1 = wr1_ref[...]
    wr2 = wr2_ref[...]

    St = S

    def make_M(w):
        Dw = jnp.transpose(D)[None, :, :] * w[:, None, :]
        return _es('bne,em->bnm', Dw, St)

    zoh = (an[..., None] == jax.lax.broadcasted_iota(
        jnp.int32, an.shape + (ZMAX,), 2)).astype(jnp.float32)
    e0 = _es('bnz,zf->bnf', zoh, embed_ref[...])
    x = jnp.pad(e0[:, :, None, :], ((0, 0), (0, 0), (0, Q - 1), (0, 0)))

    ef4 = jnp.pad(ef, ((0, 0), (1, 0)))
    efL = jnp.pad(ef4, ((0, 0), (0, L - 4)))
    efQ = jnp.concatenate([efL, efL], axis=1)
    xEF = jnp.broadcast_to(efQ[:, None, :, None], x.shape)

    qi = jax.lax.broadcasted_iota(jnp.int32, (1, 1, Q, 1), 2)
    mask_l0 = jnp.where((qi == 0) | (qi == L), 1.0, 0.0)

    for i in range(NITER):
        rs1 = basis[0] * wr1[i, 0]
        rs2 = basis[0] * wr2[i, 0]
        for k in range(1, K):
            rs1 = rs1 + basis[k] * wr1[i, k]
            rs2 = rs2 + basis[k] * wr2[i, k]

        xw1 = _es('bnqf,fg->bnqg', x, Wmp1_ref[i])
        y = _es('bnm,bmqf->bnqf', make_M(rs1), xw1)

        x00 = x[:, :, 0, :]
        xw2 = _es('bnf,fg->bng', x00, Wmp2_ref[i])
        if i < NITER - 1:
            contribs = [_es('bnm,bmf->bnf', make_M(rs2), xw2)]
            for l in range(1, L):
                contribs.append(_es('bnm,bmf->bnf',
                                           make_M(Ylist[l] * rs2), xw2))
            zeroL = jnp.zeros(x.shape[:2] + (L, F), dtype=x.dtype)
            yY = jnp.concatenate(
                [cc[:, :, None, :] for cc in contribs] + [zeroL], axis=2)
            y = y + yY
        else:
            y = y * mask_l0
            c0 = _es('bnm,bmf->bnf', make_M(rs2), xw2)
            y = y + jnp.pad(c0[:, :, None, :],
                            ((0, 0), (0, 0), (0, Q - 1), (0, 0)))

        x = x + y
        h = _gated_silu(_es('bnqf,fg->bnqg', x, Wd1_ref[i]))
        h = _es('bnqf,fg->bnqg', h, Wd2_ref[i])
        x = h + y
        x = _gated_silu(x)
        tt = x * _l0_broadcast(xEF) + xEF * _l0_broadcast(x)
        x = x + tt
        x = _es('bnqf,fg->bnqg', x, Wtd_ref[i])
        x = x + tt
        x = _gated_silu(x)
        xEF = tt

    x = _gated_silu(x)
    x0 = x[:, :, 0, :]
    for j in range(3):
        h = _es('bnf,fg->bng', x0, Wro_ref[j])
        x0 = h * _sigmoid(h)
    bias = _es('bnz,zo->bno', zoh, bias_ref[...])[..., 0]
    ae = _es('bnf,fo->bno', x0, wout_ref[...])[..., 0] + bias
    out_ref[...] = jnp.sum(ae, axis=1)[:, None, None]


def kernel(atomic_numbers, positions, Ef, dst_idx, src_idx, embed,
           element_bias, W_mp1, w_r1, W_mp2, w_r2, W_d1, W_d2, W_td, W_ro,
           w_out):
    B, N = atomic_numbers.shape
    E = dst_idx.shape[0]
    an3 = atomic_numbers.reshape(B, 1, N)
    ef3 = Ef.reshape(B, 1, 3)
    dst2 = dst_idx.reshape(E, 1).astype(jnp.int32)
    src2 = src_idx.reshape(E, 1).astype(jnp.int32)
    bias2 = element_bias.reshape(ZMAX, 1)
    wout2 = w_out.reshape(F, 1)

    grid = (B // MB,)
    full = lambda shp: pl.BlockSpec(shp, lambda i: (0,) * len(shp))
    out = pl.pallas_call(
        _mp_kernel,
        grid=grid,
        in_specs=[
            pl.BlockSpec((MB, 1, N), lambda i: (i, 0, 0)),
            pl.BlockSpec((MB, N, 3), lambda i: (i, 0, 0)),
            pl.BlockSpec((MB, 1, 3), lambda i: (i, 0, 0)),
            full((E, 1)),
            full((E, 1)),
            full((ZMAX, F)),
            full((ZMAX, 1)),
            full((NITER, F, F)),
            full((NITER, K)),
            full((NITER, F, F)),
            full((NITER, K)),
            full((NITER, F, F)),
            full((NITER, F, F)),
            full((NITER, F, F)),
            full((3, F, F)),
            full((F, 1)),
        ],
        out_specs=pl.BlockSpec((MB, 1, 1), lambda i: (i, 0, 0)),
        out_shape=jax.ShapeDtypeStruct((B, 1, 1), jnp.float32),
    )(an3, positions, ef3, dst2, src2, embed, bias2,
      W_mp1, w_r1, W_mp2, w_r2, W_d1, W_d2, W_td, W_ro, wout2)
    return out[:, 0, 0]
